# baseline (speedup 1.0000x reference)
import functools
import numpy as np
import jax
import jax.numpy as jnp
from jax import lax
from jax.experimental import pallas as pl
from jax.experimental.pallas import tpu as pltpu
from jax.experimental.pallas import tpu_sc as plsc

STRUCTURE_MASK_TOKEN = 4097
STRUCTURE_PAD_TOKEN = 4100
SEQUENCE_MASK_TOKEN = 31

KS = (1832780943, 270669613)
KC = (64467757, 2916123636)
_ROT = ((13, 15, 26, 6), (17, 29, 16, 24))


def _i32(v):
    return jnp.int32(np.uint32(v).astype(np.int32))


def _tf_bits_i32(n, k0, k1):
    k2 = np.uint32(k0) ^ np.uint32(k1) ^ np.uint32(0x1BD11BDA)
    ks = (np.uint32(k0), np.uint32(k1), k2)
    x0 = jnp.full_like(n, _i32(k0))
    x1 = n + _i32(k1)
    for i in range(5):
        for r in _ROT[i % 2]:
            x0 = x0 + x1
            x1 = (x1 << jnp.int32(r)) | lax.shift_right_logical(x1, jnp.int32(32 - r))
            x1 = x0 ^ x1
        x0 = x0 + _i32(ks[(i + 1) % 3])
        x1 = x1 + _i32(int(ks[(i + 2) % 3]) + i + 1)
    return x0 ^ x1


N_TOTAL = 4 * 8192
NW = 32
CHUNK = N_TOTAL // NW
VECS = CHUNK // 16


def _sc_body(structure_hbm, sequence_hbm, thresh_hbm,
             out_struc_hbm, out_seq_hbm,
             sv, qv, tv, osv, oqv):
    wid = lax.axis_index("s") * 2 + lax.axis_index("c")
    base = wid * CHUNK
    pltpu.sync_copy(structure_hbm.at[pl.ds(base, CHUNK)], sv)
    pltpu.sync_copy(sequence_hbm.at[pl.ds(base, CHUNK)], qv)
    pltpu.sync_copy(thresh_hbm.at[wid], tv)
    thresh = tv[...]
    lanes = lax.iota(jnp.int32, 16)

    def body(i, _):
        sl = pl.ds(i * 16, 16)
        struc = sv[sl]
        seq = qv[sl]
        n = lanes + (base + i * 16)
        bseq = _tf_bits_i32(n, *KS)
        bstr = _tf_bits_i32(n, *KC)
        fseq = lax.bitcast_convert_type(
            lax.shift_right_logical(bseq, jnp.int32(9)) | jnp.int32(0x3F800000),
            jnp.float32) - jnp.float32(1.0)
        fstr = lax.bitcast_convert_type(
            lax.shift_right_logical(bstr, jnp.int32(9)) | jnp.int32(0x3F800000),
            jnp.float32) - jnp.float32(1.0)
        pad = struc != STRUCTURE_PAD_TOKEN
        osv[sl] = jnp.where((fstr < thresh) & pad, STRUCTURE_MASK_TOKEN, struc)
        oqv[sl] = jnp.where((fseq < thresh) & pad, SEQUENCE_MASK_TOKEN, seq)
        return 0

    lax.fori_loop(0, VECS, body, 0)
    pltpu.sync_copy(osv, out_struc_hbm.at[pl.ds(base, CHUNK)])
    pltpu.sync_copy(oqv, out_seq_hbm.at[pl.ds(base, CHUNK)])


def kernel(structure, sequence, t):
    B, L = structure.shape
    thr = jnp.tile((jnp.float32(1.0) - t)[:, None], (1, 8 * 16)).reshape(NW, 16)
    mesh = plsc.VectorSubcoreMesh(core_axis_name="c", subcore_axis_name="s")
    k = pl.kernel(
        _sc_body,
        out_type=(
            jax.ShapeDtypeStruct((N_TOTAL,), jnp.int32),
            jax.ShapeDtypeStruct((N_TOTAL,), jnp.int32),
        ),
        mesh=mesh,
        scratch_types=[
            pltpu.VMEM((CHUNK,), jnp.int32),
            pltpu.VMEM((CHUNK,), jnp.int32),
            pltpu.VMEM((16,), jnp.float32),
            pltpu.VMEM((CHUNK,), jnp.int32),
            pltpu.VMEM((CHUNK,), jnp.int32),
        ],
    )
    out_struc, out_seq = k(structure.reshape(-1), sequence.reshape(-1), thr)
    return (out_struc.reshape(B, L), out_seq.reshape(B, L), t)


# speedup vs baseline: 1.1235x; 1.1235x over previous
import functools
import numpy as np
import jax
import jax.numpy as jnp
from jax import lax
from jax.experimental import pallas as pl
from jax.experimental.pallas import tpu as pltpu
from jax.experimental.pallas import tpu_sc as plsc

STRUCTURE_MASK_TOKEN = 4097
STRUCTURE_PAD_TOKEN = 4100
SEQUENCE_MASK_TOKEN = 31

KS = (1832780943, 270669613)
KC = (64467757, 2916123636)
_ROT = ((13, 15, 26, 6), (17, 29, 16, 24))


def _i32(v):
    return jnp.int32(np.uint32(v).astype(np.int32))


def _tf_bits_i32(n, k0, k1):
    k2 = np.uint32(k0) ^ np.uint32(k1) ^ np.uint32(0x1BD11BDA)
    ks = (np.uint32(k0), np.uint32(k1), k2)
    x0 = jnp.full_like(n, _i32(k0))
    x1 = n + _i32(k1)
    for i in range(5):
        for r in _ROT[i % 2]:
            x0 = x0 + x1
            x1 = (x1 << jnp.int32(r)) | lax.shift_right_logical(x1, jnp.int32(32 - r))
            x1 = x0 ^ x1
        x0 = x0 + _i32(ks[(i + 1) % 3])
        x1 = x1 + _i32(int(ks[(i + 2) % 3]) + i + 1)
    return x0 ^ x1


N_TOTAL = 4 * 8192
NW = 32
CHUNK = N_TOTAL // NW
VECS = CHUNK // 16


def _sc_body(structure_hbm, sequence_hbm, thresh_hbm,
             out_struc_hbm, out_seq_hbm,
             sv, qv, tv, osv, oqv):
    wid = lax.axis_index("s") * 2 + lax.axis_index("c")
    base = wid * CHUNK
    pltpu.sync_copy(structure_hbm.at[pl.ds(base, CHUNK)], sv)
    pltpu.sync_copy(sequence_hbm.at[pl.ds(base, CHUNK)], qv)
    pltpu.sync_copy(thresh_hbm.at[wid], tv)
    thresh = tv[...]
    lanes = lax.iota(jnp.int32, 16)

    @functools.partial(plsc.parallel_loop, 0, VECS, unroll=8)
    def body(i):
        sl = pl.ds(i * 16, 16)
        struc = sv[sl]
        seq = qv[sl]
        n = lanes + (base + i * 16)
        bseq = _tf_bits_i32(n, *KS)
        bstr = _tf_bits_i32(n, *KC)
        fseq = lax.bitcast_convert_type(
            lax.shift_right_logical(bseq, jnp.int32(9)) | jnp.int32(0x3F800000),
            jnp.float32) - jnp.float32(1.0)
        fstr = lax.bitcast_convert_type(
            lax.shift_right_logical(bstr, jnp.int32(9)) | jnp.int32(0x3F800000),
            jnp.float32) - jnp.float32(1.0)
        pad = struc != STRUCTURE_PAD_TOKEN
        osv[sl] = jnp.where((fstr < thresh) & pad, STRUCTURE_MASK_TOKEN, struc)
        oqv[sl] = jnp.where((fseq < thresh) & pad, SEQUENCE_MASK_TOKEN, seq)
    pltpu.sync_copy(osv, out_struc_hbm.at[pl.ds(base, CHUNK)])
    pltpu.sync_copy(oqv, out_seq_hbm.at[pl.ds(base, CHUNK)])


def kernel(structure, sequence, t):
    B, L = structure.shape
    thr = jnp.tile((jnp.float32(1.0) - t)[:, None], (1, 8 * 16)).reshape(NW, 16)
    mesh = plsc.VectorSubcoreMesh(core_axis_name="c", subcore_axis_name="s")
    k = pl.kernel(
        _sc_body,
        out_type=(
            jax.ShapeDtypeStruct((N_TOTAL,), jnp.int32),
            jax.ShapeDtypeStruct((N_TOTAL,), jnp.int32),
        ),
        mesh=mesh,
        scratch_types=[
            pltpu.VMEM((CHUNK,), jnp.int32),
            pltpu.VMEM((CHUNK,), jnp.int32),
            pltpu.VMEM((16,), jnp.float32),
            pltpu.VMEM((CHUNK,), jnp.int32),
            pltpu.VMEM((CHUNK,), jnp.int32),
        ],
    )
    out_struc, out_seq = k(structure.reshape(-1), sequence.reshape(-1), thr)
    return (out_struc.reshape(B, L), out_seq.reshape(B, L), t)
